# baseline (speedup 1.0000x reference)
import math

import jax
import jax.numpy as jnp
from jax.experimental import pallas as pl

B, T, V = 16, 150, 25
TV = T * V
EPS = 1e-5
NEG = 0.05
NSTAT = float(B * T * V)

CHUNK_T = 6
L = CHUNK_T * V
NCH = T // CHUNK_T

F32 = jnp.float32


def _mean_proj():
    r = jax.lax.broadcasted_iota(jnp.int32, (TV, V), 0) % V
    c = jax.lax.broadcasted_iota(jnp.int32, (TV, V), 1)
    return jnp.where(r == c, 1.0 / T, 0.0).astype(F32)


def _dot(a, b, dims):
    return jax.lax.dot_general(a, b, (dims, ((), ())),
                               preferred_element_type=F32)


def _adjacency(xf, cin):
    e = _dot(xf, _mean_proj(), ((1,), (0,)))
    logits = _dot(e, e, ((0,), (0,))) * (1.0 / math.sqrt(float(cin)))
    m = jnp.max(logits, axis=-1, keepdims=True)
    ex = jnp.exp(logits - m)
    return ex / jnp.sum(ex, axis=-1, keepdims=True)


def _mix_mat(adj):
    i = jax.lax.broadcasted_iota(jnp.int32, (L, V), 0)
    k = jax.lax.broadcasted_iota(jnp.int32, (L, V), 1)
    s = jnp.where(i % V == k, 1.0, 0.0).astype(F32)
    sa = _dot(s, adj, ((1,), (0,)))
    m0 = _dot(sa, s, ((1,), (1,)))
    r = jax.lax.broadcasted_iota(jnp.int32, (L, L), 0) // V
    c = jax.lax.broadcasted_iota(jnp.int32, (L, L), 1) // V
    return jnp.where(r == c, m0, 0.0)


def _layer(xf, w, b, cin, y_ref, st_ref):
    adj = _adjacency(xf, cin)
    h = _dot(w, xf, ((1,), (0,))) + b
    mix = _mix_mat(adj)
    cout = h.shape[0]
    s = jnp.zeros((cout, 1), dtype=F32)
    q = jnp.zeros((cout, 1), dtype=F32)
    for kk in range(NCH):
        sl = slice(kk * L, (kk + 1) * L)
        yc = _dot(h[:, sl], mix, ((1,), (0,)))
        y_ref[0, :, sl] = yc
        if st_ref is not None:
            s = s + jnp.sum(yc, axis=1, keepdims=True)
            q = q + jnp.sum(yc * yc, axis=1, keepdims=True)
    if st_ref is not None:
        st_ref[0, :, 0:1] = s
        st_ref[0, :, 1:2] = q


def _bn_leaky(y, st, g, be):
    tot = jnp.sum(st, axis=0)
    mean = tot[:, 0:1] / NSTAT
    var = tot[:, 1:2] / NSTAT - mean * mean
    inv = jax.lax.rsqrt(var + EPS)
    xh = (y - mean) * inv * g + be
    return jnp.where(xh >= 0, xh, NEG * xh)


def _k1(x_ref, w_ref, b_ref, y_ref, st_ref):
    _layer(x_ref[0], w_ref[...], b_ref[...], 193, y_ref, st_ref)


def _k2(y1_ref, st_ref, g_ref, be_ref, w_ref, b_ref, y_ref, st2_ref):
    x2 = _bn_leaky(y1_ref[0], st_ref[...], g_ref[...], be_ref[...])
    _layer(x2, w_ref[...], b_ref[...], 128, y_ref, st2_ref)


def _k3(y2_ref, st_ref, g_ref, be_ref, w_ref, b_ref, y_ref):
    x3 = _bn_leaky(y2_ref[0], st_ref[...], g_ref[...], be_ref[...])
    _layer(x3, w_ref[...], b_ref[...], 64, y_ref, None)


def _full(shape):
    return pl.BlockSpec(shape, lambda b: (0,) * len(shape))


def _perb(c):
    return pl.BlockSpec((1, c, TV), lambda b: (b, 0, 0))


def _stspec(c):
    return pl.BlockSpec((1, c, 2), lambda b: (b, 0, 0))


def kernel(x, W1, b1, g1, be1, W2, b2, g2, be2, W3, b3):
    x2d = x.reshape(B, 193, TV)

    y1, st1 = pl.pallas_call(
        _k1,
        grid=(B,),
        in_specs=[_perb(193), _full((128, 193)), _full((128, 1))],
        out_specs=[_perb(128), _stspec(128)],
        out_shape=[jax.ShapeDtypeStruct((B, 128, TV), F32),
                   jax.ShapeDtypeStruct((B, 128, 2), F32)],
    )(x2d, W1, b1.reshape(128, 1))

    y2, st2 = pl.pallas_call(
        _k2,
        grid=(B,),
        in_specs=[_perb(128), _full((B, 128, 2)), _full((128, 1)),
                  _full((128, 1)), _full((64, 128)), _full((64, 1))],
        out_specs=[_perb(64), _stspec(64)],
        out_shape=[jax.ShapeDtypeStruct((B, 64, TV), F32),
                   jax.ShapeDtypeStruct((B, 64, 2), F32)],
    )(y1, st1, g1.reshape(128, 1), be1.reshape(128, 1), W2,
      b2.reshape(64, 1))

    y3 = pl.pallas_call(
        _k3,
        grid=(B,),
        in_specs=[_perb(64), _full((B, 64, 2)), _full((64, 1)),
                  _full((64, 1)), _full((3, 64)), _full((3, 1))],
        out_specs=_perb(3),
        out_shape=jax.ShapeDtypeStruct((B, 3, TV), F32),
    )(y2, st2, g2.reshape(64, 1), be2.reshape(64, 1), W3,
      b3.reshape(3, 1))

    return y3.reshape(B, 3, T, V)
